# Initial kernel scaffold; baseline (speedup 1.0000x reference)
import jax
import jax.numpy as jnp
from jax.experimental import pallas as pl


def kernel(x, y, labels, labels_2, enc_params, clb_enc_params, dec_params, clb_dec_params, codebooks):
    raise NotImplementedError("write your pallas kernel here")


# speedup vs baseline: 1.1277x; 1.1277x over previous
import functools

import jax
import jax.numpy as jnp
from jax.experimental import pallas as pl
from jax.experimental.pallas import tpu as pltpu

_B = 16384
_IN_DIM = 768
_CLB_DIM = 32
_E_DIM = 64
_NUM_CODES = 1024
_NUM_LEVELS = 4
_BETA = 0.25
_TILE = 512

_PREC = jax.lax.Precision.DEFAULT


def _mlp(h, ws, bs):
    n = len(ws)
    for i in range(n):
        h = jnp.dot(h, ws[i][...], precision=_PREC) + bs[i][...]
        if i < n - 1:
            h = jax.nn.relu(h)
    return h


def _rq(r, cb_ref, idx_ref, loss_ref, loss_row0):
    acc = jnp.zeros_like(r)
    for l in range(_NUM_LEVELS):
        emb = cb_ref[l]
        hi = emb.astype(jnp.bfloat16).astype(jnp.float32)
        rem = emb - hi
        mid = rem.astype(jnp.bfloat16).astype(jnp.float32)
        lo = rem - mid
        esq = jnp.sum(emb * emb, axis=1)
        prod = jax.lax.dot_general(r, emb, (((1,), (1,)), ((), ())),
                                   precision=_PREC)
        d = jnp.sum(r * r, axis=1, keepdims=True) - 2.0 * prod + esq[None, :]
        lanes = jax.lax.broadcasted_iota(jnp.int32, d.shape, 1)
        dmin = jnp.min(d, axis=1, keepdims=True)
        ind = jnp.min(jnp.where(d == dmin, lanes, jnp.int32(2**30)), axis=1)
        oh = (lanes == ind[:, None]).astype(jnp.float32)
        xq = (jnp.dot(oh, hi, precision=_PREC)
              + jnp.dot(oh, mid, precision=_PREC)
              + jnp.dot(oh, lo, precision=_PREC))
        diff = xq - r
        sse = jnp.sum(diff * diff)
        loss_ref[0:1, loss_row0 + l:loss_row0 + l + 1, :] = jnp.full(
            (1, 1, 128), sse, jnp.float32)
        xq_st = r + diff
        r = r - xq_st
        acc = acc + xq_st
        idx_ref[:, l:l + 1] = ind[:, None]
    return acc


def _body(x_ref, y_ref,
          ew0, ew1, ew2, ew3, eb0, eb1, eb2, eb3,
          cw0, cw1, cw2, cw3, cb0, cb1, cb2, cb3,
          dw0, dw1, dw2, dw3, db0, db1, db2, db3,
          gw0, gw1, gw2, gw3, gb0, gb1, gb2, gb3,
          cb_ref,
          out_ref, out_clb_ref, xq_ref, yq_ref, idx_ref, idx2_ref, loss_ref):
    x_e = _mlp(x_ref[...], (ew0, ew1, ew2, ew3), (eb0, eb1, eb2, eb3))
    x_q = _rq(x_e, cb_ref, idx_ref, loss_ref, 0)
    xq_ref[...] = x_q
    out_ref[...] = _mlp(x_q, (dw0, dw1, dw2, dw3), (db0, db1, db2, db3))

    y_e = _mlp(y_ref[...], (cw0, cw1, cw2, cw3), (cb0, cb1, cb2, cb3))
    y_q = _rq(y_e, cb_ref, idx2_ref, loss_ref, _NUM_LEVELS)
    yq_ref[...] = y_q
    out_clb_ref[...] = _mlp(y_e, (gw0, gw1, gw2, gw3), (gb0, gb1, gb2, gb3))


def kernel(x, y, labels, labels_2, enc_params, clb_enc_params, dec_params,
           clb_dec_params, codebooks):
    del labels, labels_2
    nt = _B // _TILE
    cb = jnp.stack(codebooks)

    def _wb(ps):
        ws = [ps[2 * i] for i in range(4)]
        bs = [ps[2 * i + 1].reshape(1, -1) for i in range(4)]
        return ws, bs

    ews, ebs = _wb(enc_params)
    cws, cbs = _wb(clb_enc_params)
    dws, dbs = _wb(dec_params)
    gws, gbs = _wb(clb_dec_params)

    def _full(a):
        nd = a.ndim
        return pl.BlockSpec(a.shape, lambda i, _n=nd: (0,) * _n)

    row = lambda d: pl.BlockSpec((_TILE, d), lambda i: (i, 0))
    weights = ews + ebs + cws + cbs + dws + dbs + gws + gbs + [cb]

    out_shapes = (
        jax.ShapeDtypeStruct((_B, _IN_DIM), jnp.float32),
        jax.ShapeDtypeStruct((_B, _CLB_DIM), jnp.float32),
        jax.ShapeDtypeStruct((_B, _E_DIM), jnp.float32),
        jax.ShapeDtypeStruct((_B, _E_DIM), jnp.float32),
        jax.ShapeDtypeStruct((_B, _NUM_LEVELS), jnp.int32),
        jax.ShapeDtypeStruct((_B, _NUM_LEVELS), jnp.int32),
        jax.ShapeDtypeStruct((nt, 2 * _NUM_LEVELS, 128), jnp.float32),
    )
    out_specs = (
        row(_IN_DIM), row(_CLB_DIM), row(_E_DIM), row(_E_DIM),
        row(_NUM_LEVELS), row(_NUM_LEVELS),
        pl.BlockSpec((1, 2 * _NUM_LEVELS, 128), lambda i: (i, 0, 0)),
    )

    out, out_clb, x_q, y_q, indices, indices_2, loss_parts = pl.pallas_call(
        _body,
        grid=(nt,),
        in_specs=[row(_IN_DIM), row(_CLB_DIM)] + [_full(w) for w in weights],
        out_specs=out_specs,
        out_shape=out_shapes,
    )(x, y, *weights)

    sse = jnp.sum(loss_parts[:, :, 0], axis=0)
    m = sse / jnp.float32(_B * _E_DIM)
    per_level = m + _BETA * m
    rq_loss = jnp.mean(per_level[:_NUM_LEVELS])
    rq_loss_2 = jnp.mean(per_level[_NUM_LEVELS:])
    return (out, out_clb, rq_loss, rq_loss_2, indices, indices_2, x_q, y_q)


# speedup vs baseline: 1.1676x; 1.0354x over previous
import jax
import jax.numpy as jnp
from jax.experimental import pallas as pl

_B = 16384
_IN_DIM = 768
_CLB_DIM = 32
_E_DIM = 64
_NUM_CODES = 1024
_NUM_LEVELS = 4
_BETA = 0.25
_TILE = 512

_PREC = jax.lax.Precision.DEFAULT


def _mlp(h, ws, bs):
    n = len(ws)
    for i in range(n):
        h = jnp.dot(h, ws[i][...], precision=_PREC) + bs[i][...]
        if i < n - 1:
            h = jax.nn.relu(h)
    return h


def _rq(r, cb_ref, gat_ref, esq_ref, idx_ref, idx2_ref, loss_ref):
    t = _TILE
    acc = jnp.zeros_like(r)
    for l in range(_NUM_LEVELS):
        emb = cb_ref[l]
        prod = jax.lax.dot_general(r, emb, (((1,), (1,)), ((), ())),
                                   precision=_PREC)
        d = jnp.sum(r * r, axis=1, keepdims=True) - 2.0 * prod + esq_ref[l]
        lanes = jax.lax.broadcasted_iota(jnp.int32, d.shape, 1)
        dmin = jnp.min(d, axis=1, keepdims=True)
        ind = jnp.min(jnp.where(d == dmin, lanes, jnp.int32(2**30)), axis=1)
        oh = (lanes == ind[:, None]).astype(jnp.float32)
        hi = emb.astype(jnp.bfloat16).astype(jnp.float32)
        rem = emb - hi
        mid = rem.astype(jnp.bfloat16).astype(jnp.float32)
        lo = rem - mid
        xq = ((jnp.dot(oh, hi, precision=_PREC)
               + jnp.dot(oh, mid, precision=_PREC))
              + jnp.dot(oh, lo, precision=_PREC))
        diff = xq - r
        dx, dy = diff[:t], diff[t:]
        loss_ref[0:1, l:l + 1, :] = jnp.full(
            (1, 1, 128), jnp.sum(dx * dx), jnp.float32)
        loss_ref[0:1, _NUM_LEVELS + l:_NUM_LEVELS + l + 1, :] = jnp.full(
            (1, 1, 128), jnp.sum(dy * dy), jnp.float32)
        xq_st = r + diff
        r = r - xq_st
        acc = acc + xq_st
        idx_ref[:, l:l + 1] = ind[:t, None]
        idx2_ref[:, l:l + 1] = ind[t:, None]
    return acc


def _body(x_ref, y_ref,
          ew0, ew1, ew2, ew3, eb0, eb1, eb2, eb3,
          cw0, cw1, cw2, cw3, cb0, cb1, cb2, cb3,
          dw0, dw1, dw2, dw3, db0, db1, db2, db3,
          gw0, gw1, gw2, gw3, gb0, gb1, gb2, gb3,
          cb_ref, gat_ref, esq_ref,
          out_ref, out_clb_ref, xq_ref, yq_ref, idx_ref, idx2_ref, loss_ref):
    t = _TILE
    x_e = _mlp(x_ref[...], (ew0, ew1, ew2, ew3), (eb0, eb1, eb2, eb3))
    y_e = _mlp(y_ref[...], (cw0, cw1, cw2, cw3), (cb0, cb1, cb2, cb3))
    q = _rq(jnp.concatenate([x_e, y_e], axis=0), cb_ref, gat_ref, esq_ref,
            idx_ref, idx2_ref, loss_ref)
    x_q, y_q = q[:t], q[t:]
    xq_ref[...] = x_q
    yq_ref[...] = y_q
    out_ref[...] = _mlp(x_q, (dw0, dw1, dw2, dw3), (db0, db1, db2, db3))
    out_clb_ref[...] = _mlp(y_e, (gw0, gw1, gw2, gw3), (gb0, gb1, gb2, gb3))


def kernel(x, y, labels, labels_2, enc_params, clb_enc_params, dec_params,
           clb_dec_params, codebooks):
    del labels, labels_2
    nt = _B // _TILE
    cb = jnp.stack(codebooks)
    hi = cb.astype(jnp.bfloat16)
    rem = cb - hi.astype(jnp.float32)
    mid = rem.astype(jnp.bfloat16)
    lo = (rem - mid.astype(jnp.float32)).astype(jnp.bfloat16)
    gat = jnp.concatenate([hi, mid, lo], axis=-1)
    esq = jnp.stack([jnp.sum(e ** 2, axis=1)[None, :] for e in codebooks])

    def _wb(ps):
        ws = [ps[2 * i] for i in range(4)]
        bs = [ps[2 * i + 1].reshape(1, -1) for i in range(4)]
        return ws, bs

    ews, ebs = _wb(enc_params)
    cws, cbs = _wb(clb_enc_params)
    dws, dbs = _wb(dec_params)
    gws, gbs = _wb(clb_dec_params)

    def _full(a):
        nd = a.ndim
        return pl.BlockSpec(a.shape, lambda i, _n=nd: (0,) * _n)

    row = lambda d: pl.BlockSpec((_TILE, d), lambda i: (i, 0))
    weights = ews + ebs + cws + cbs + dws + dbs + gws + gbs + [cb, gat, esq]

    out_shapes = (
        jax.ShapeDtypeStruct((_B, _IN_DIM), jnp.float32),
        jax.ShapeDtypeStruct((_B, _CLB_DIM), jnp.float32),
        jax.ShapeDtypeStruct((_B, _E_DIM), jnp.float32),
        jax.ShapeDtypeStruct((_B, _E_DIM), jnp.float32),
        jax.ShapeDtypeStruct((_B, _NUM_LEVELS), jnp.int32),
        jax.ShapeDtypeStruct((_B, _NUM_LEVELS), jnp.int32),
        jax.ShapeDtypeStruct((nt, 2 * _NUM_LEVELS, 128), jnp.float32),
    )
    out_specs = (
        row(_IN_DIM), row(_CLB_DIM), row(_E_DIM), row(_E_DIM),
        row(_NUM_LEVELS), row(_NUM_LEVELS),
        pl.BlockSpec((1, 2 * _NUM_LEVELS, 128), lambda i: (i, 0, 0)),
    )

    out, out_clb, x_q, y_q, indices, indices_2, loss_parts = pl.pallas_call(
        _body,
        grid=(nt,),
        in_specs=[row(_IN_DIM), row(_CLB_DIM)] + [_full(w) for w in weights],
        out_specs=out_specs,
        out_shape=out_shapes,
    )(x, y, *weights)

    sse = jnp.sum(loss_parts[:, :, 0], axis=0)
    m = sse / jnp.float32(_B * _E_DIM)
    per_level = m + _BETA * m
    rq_loss = jnp.mean(per_level[:_NUM_LEVELS])
    rq_loss_2 = jnp.mean(per_level[_NUM_LEVELS:])
    return (out, out_clb, rq_loss, rq_loss_2, indices, indices_2, x_q, y_q)


# speedup vs baseline: 1.7706x; 1.5164x over previous
import jax
import jax.numpy as jnp
from jax.experimental import pallas as pl
from jax.experimental.pallas import tpu as pltpu

_B = 16384
_IN_DIM = 768
_CLB_DIM = 32
_E_DIM = 64
_NUM_CODES = 1024
_NUM_LEVELS = 4
_BETA = 0.25
_TILE = 512

_PREC = jax.lax.Precision.DEFAULT


def _mlp(h, ws, bs):
    n = len(ws)
    for i in range(n):
        h = jnp.dot(h, ws[i][...], precision=_PREC) + bs[i][...]
        if i < n - 1:
            h = jax.nn.relu(h)
    return h


def _rq(r, cb_ref, gat_ref, esq_ref, idx_ref, idx2_ref, loss_ref):
    t = _TILE
    acc = jnp.zeros_like(r)
    for l in range(_NUM_LEVELS):
        emb = cb_ref[l]
        prod = jax.lax.dot_general(r, emb, (((1,), (1,)), ((), ())),
                                   precision=_PREC)
        d = jnp.sum(r * r, axis=1, keepdims=True) - 2.0 * prod + esq_ref[l]
        lanes = jax.lax.broadcasted_iota(jnp.int32, d.shape, 1)
        dmin = jnp.min(d, axis=1, keepdims=True)
        ind = jnp.min(jnp.where(d == dmin, lanes, jnp.int32(2**30)), axis=1)
        oh = (lanes == ind[:, None]).astype(jnp.float32)
        s = jnp.dot(oh, gat_ref[l], precision=_PREC)
        xq = (s[:, :_E_DIM] + s[:, _E_DIM:2 * _E_DIM]) + s[:, 2 * _E_DIM:]
        diff = xq - r
        dx, dy = diff[:t], diff[t:]
        loss_ref[0:1, l:l + 1, :] = jnp.full(
            (1, 1, 128), jnp.sum(dx * dx), jnp.float32)
        loss_ref[0:1, _NUM_LEVELS + l:_NUM_LEVELS + l + 1, :] = jnp.full(
            (1, 1, 128), jnp.sum(dy * dy), jnp.float32)
        xq_st = r + diff
        r = r - xq_st
        acc = acc + xq_st
        idx_ref[:, l:l + 1] = ind[:t, None]
        idx2_ref[:, l:l + 1] = ind[t:, None]
    return acc


def _body(x_ref, y_ref,
          ew0, ew1, ew2, ew3, eb0, eb1, eb2, eb3,
          cw0, cw1, cw2, cw3, cb0, cb1, cb2, cb3,
          dw0, dw1, dw2, dw3, db0, db1, db2, db3,
          gw0, gw1, gw2, gw3, gb0, gb1, gb2, gb3,
          cb_ref, esq_ref,
          out_ref, out_clb_ref, xq_ref, yq_ref, idx_ref, idx2_ref, loss_ref,
          gat_ref):
    t = _TILE
    @pl.when(pl.program_id(0) == 0)
    def _fill_gat():
        cb4 = cb_ref[...]
        hi = cb4.astype(jnp.bfloat16).astype(jnp.float32)
        rem = cb4 - hi
        mid = rem.astype(jnp.bfloat16).astype(jnp.float32)
        lo = rem - mid
        gat_ref[:, :, 0:_E_DIM] = hi
        gat_ref[:, :, _E_DIM:2 * _E_DIM] = mid
        gat_ref[:, :, 2 * _E_DIM:] = lo

    x_e = _mlp(x_ref[...], (ew0, ew1, ew2, ew3), (eb0, eb1, eb2, eb3))
    y_e = _mlp(y_ref[...], (cw0, cw1, cw2, cw3), (cb0, cb1, cb2, cb3))
    q = _rq(jnp.concatenate([x_e, y_e], axis=0), cb_ref, gat_ref, esq_ref,
            idx_ref, idx2_ref, loss_ref)
    x_q, y_q = q[:t], q[t:]
    xq_ref[...] = x_q
    yq_ref[...] = y_q
    out_ref[...] = _mlp(x_q, (dw0, dw1, dw2, dw3), (db0, db1, db2, db3))
    out_clb_ref[...] = _mlp(y_e, (gw0, gw1, gw2, gw3), (gb0, gb1, gb2, gb3))


def kernel(x, y, labels, labels_2, enc_params, clb_enc_params, dec_params,
           clb_dec_params, codebooks):
    del labels, labels_2
    nt = _B // _TILE
    cb = jnp.stack(codebooks)
    esq = jnp.stack([jnp.sum(e ** 2, axis=1)[None, :] for e in codebooks])

    def _wb(ps):
        ws = [ps[2 * i] for i in range(4)]
        bs = [ps[2 * i + 1].reshape(1, -1) for i in range(4)]
        return ws, bs

    ews, ebs = _wb(enc_params)
    cws, cbs = _wb(clb_enc_params)
    dws, dbs = _wb(dec_params)
    gws, gbs = _wb(clb_dec_params)

    def _full(a):
        nd = a.ndim
        return pl.BlockSpec(a.shape, lambda i, _n=nd: (0,) * _n)

    row = lambda d: pl.BlockSpec((_TILE, d), lambda i: (i, 0))
    weights = ews + ebs + cws + cbs + dws + dbs + gws + gbs + [cb, esq]

    out_shapes = (
        jax.ShapeDtypeStruct((_B, _IN_DIM), jnp.float32),
        jax.ShapeDtypeStruct((_B, _CLB_DIM), jnp.float32),
        jax.ShapeDtypeStruct((_B, _E_DIM), jnp.float32),
        jax.ShapeDtypeStruct((_B, _E_DIM), jnp.float32),
        jax.ShapeDtypeStruct((_B, _NUM_LEVELS), jnp.int32),
        jax.ShapeDtypeStruct((_B, _NUM_LEVELS), jnp.int32),
        jax.ShapeDtypeStruct((nt, 2 * _NUM_LEVELS, 128), jnp.float32),
    )
    out_specs = (
        row(_IN_DIM), row(_CLB_DIM), row(_E_DIM), row(_E_DIM),
        row(_NUM_LEVELS), row(_NUM_LEVELS),
        pl.BlockSpec((1, 2 * _NUM_LEVELS, 128), lambda i: (i, 0, 0)),
    )

    out, out_clb, x_q, y_q, indices, indices_2, loss_parts = pl.pallas_call(
        _body,
        grid=(nt,),
        in_specs=[row(_IN_DIM), row(_CLB_DIM)] + [_full(w) for w in weights],
        out_specs=out_specs,
        out_shape=out_shapes,
        scratch_shapes=[
            pltpu.VMEM((_NUM_LEVELS, _NUM_CODES, 3 * _E_DIM), jnp.float32)],
    )(x, y, *weights)

    sse = jnp.sum(loss_parts[:, :, 0], axis=0)
    m = sse / jnp.float32(_B * _E_DIM)
    per_level = m + _BETA * m
    rq_loss = jnp.mean(per_level[:_NUM_LEVELS])
    rq_loss_2 = jnp.mean(per_level[_NUM_LEVELS:])
    return (out, out_clb, rq_loss, rq_loss_2, indices, indices_2, x_q, y_q)
